# Initial kernel scaffold; baseline (speedup 1.0000x reference)
import jax
import jax.numpy as jnp
from jax.experimental import pallas as pl


def kernel(pred, det_bboxes, cls_scores, labels):
    raise NotImplementedError("write your pallas kernel here")


# speedup vs baseline: 46.2992x; 46.2992x over previous
import jax
import jax.numpy as jnp
from jax.experimental import pallas as pl

N = 2000
CH = 5
HMAP = 56
HW = HMAP * HMAP
NCH = 4
TH = 0.2
DET = 150

BS1 = 200
BS2 = 400


def _heat_kernel(pred_ref, det_ref, maxv_ref, feat_ref):
    det = det_ref[...]
    ms, idxs = [], []
    for c in range(NCH):
        xc = pred_ref[:, c * HW:(c + 1) * HW]
        mc = jnp.max(xc, axis=1, keepdims=True)
        pos = jax.lax.broadcasted_iota(jnp.int32, xc.shape, 1)
        idxc = jnp.min(jnp.where(xc == mc, pos, HW), axis=1, keepdims=True)
        ms.append(mc)
        idxs.append(idxc)
    m = jnp.concatenate(ms, axis=1)
    idx = jnp.concatenate(idxs, axis=1)
    xsf = (idx % HMAP).astype(jnp.float32)
    ysf = (idx // HMAP).astype(jnp.float32)
    widths = det[:, 2:3] - det[:, 0:1]
    heights = det[:, 3:4] - det[:, 1:2]
    x1 = det[:, 0:1] - widths / 2
    y1 = det[:, 1:2] - heights / 2
    abs_xs = (xsf + 0.5) / HMAP * widths * 2 + x1
    abs_ys = (ysf + 0.5) / HMAP * heights * 2 + y1
    bx1 = jnp.min(abs_xs, axis=1, keepdims=True)
    bx2 = jnp.max(abs_xs, axis=1, keepdims=True)
    by1 = jnp.min(abs_ys, axis=1, keepdims=True)
    by2 = jnp.max(abs_ys, axis=1, keepdims=True)
    maxv_ref[...] = m
    feat_ref[...] = jnp.concatenate(
        [abs_xs, abs_ys, bx1, bx2, by1, by2], axis=1)


def _rank_kernel(feat_ref, featT_ref, out_ref):
    pid = pl.program_id(0)
    s_j = feat_ref[:, 12:13]
    s_i = featT_ref[12:13, :]
    jglob = jax.lax.broadcasted_iota(jnp.int32, (BS2, 1), 0) + pid * BS2
    iglob = jax.lax.broadcasted_iota(jnp.int32, (1, N), 1)
    before = (s_j > s_i) | ((s_j == s_i) & (jglob < iglob))
    part = jnp.sum(jnp.where(before, 1.0, 0.0), axis=0, keepdims=True)

    @pl.when(pid == 0)
    def _():
        out_ref[...] = jnp.zeros_like(out_ref)

    out_ref[...] += part.reshape(1, 1, N)


def _permute_kernel(rank_ref, feat_ref, sd_ref):
    pid = pl.program_id(0)
    rank_row = rank_ref[...].reshape(1, N)
    p_glob = (jax.lax.broadcasted_iota(jnp.int32, (BS2, 1), 0)
              + pid * BS2).astype(jnp.float32)
    onehot = jnp.where(rank_row == p_glob, 1.0, 0.0)
    sd_ref[...] = jnp.dot(onehot, feat_ref[...],
                          preferred_element_type=jnp.float32)


def _mt_kernel(sd_ref, sdt_ref, mt_ref):
    pid = pl.program_id(0)
    sd = sd_ref[...]
    bx1i, bx2i = sd[:, 8:9], sd[:, 9:10]
    by1i, by2i = sd[:, 10:11], sd[:, 11:12]
    labi = sd[:, 13:14]
    bx1j = sdt_ref[8:9, :]
    bx2j = sdt_ref[9:10, :]
    by1j = sdt_ref[10:11, :]
    by2j = sdt_ref[11:12, :]
    labj = sdt_ref[13:14, :]
    ix1 = jnp.maximum(bx1i, bx1j)
    iy1 = jnp.maximum(by1i, by1j)
    ix2 = jnp.minimum(bx2i, bx2j)
    iy2 = jnp.minimum(by2i, by2j)
    inter = jnp.maximum(ix2 - ix1, 0.0) * jnp.maximum(iy2 - iy1, 0.0)
    ai = jnp.maximum(bx2i - bx1i, 0.0) * jnp.maximum(by2i - by1i, 0.0)
    aj = jnp.maximum(bx2j - bx1j, 0.0) * jnp.maximum(by2j - by1j, 0.0)
    union = ai + aj - inter
    iou = inter / jnp.maximum(union, 1e-9)
    iglob = jax.lax.broadcasted_iota(jnp.int32, (BS2, 1), 0) + pid * BS2
    jglob = jax.lax.broadcasted_iota(jnp.int32, (1, N), 1)
    sup = (iou > TH) & (labi == labj) & (jglob < iglob)
    mt_ref[...] = jnp.where(sup, 1.0, 0.0)


def _fix_kernel(mt_ref, sd_ref, out_ref):
    mt = mt_ref[...]
    sd = sd_ref[...]
    s = sd[:, 12:13]
    lab = sd[:, 13:14]
    k0 = jnp.where(lab > 0.0, 1.0, 0.0)

    def step(k):
        amt = jnp.dot(mt, k, preferred_element_type=jnp.float32)
        return k0 * jnp.where(amt == 0.0, 1.0, 0.0)

    k1 = step(k0)

    def cond(c):
        return c[1]

    def body(c):
        k, _ = c
        k2 = step(k)
        return k2, jnp.any(k2 != k)

    k, _ = jax.lax.while_loop(cond, body, (k1, jnp.any(k1 != k0)))

    ic = jax.lax.broadcasted_iota(jnp.int32, (N, 1), 0)
    jr = jax.lax.broadcasted_iota(jnp.int32, (1, N), 1)
    lle = jnp.where(jr <= ic, 1.0, 0.0)
    cnt = jnp.dot(lle, k, preferred_element_type=jnp.float32)
    sel = (k > 0.0) & (cnt == jnp.float32(DET))
    th = jnp.max(jnp.where(sel, s, -jnp.inf))
    kf = (k > 0.0) & (s >= th)
    fs = jnp.where(kf, s, 0.0)
    out_ref[...] = jnp.concatenate(
        [sd[:, 0:1], sd[:, 4:5], sd[:, 1:2], sd[:, 5:6],
         sd[:, 2:3], sd[:, 6:7], sd[:, 3:4], sd[:, 7:8], fs], axis=1)


def kernel(pred, det_bboxes, cls_scores, labels):
    pred2 = pred.reshape(N, CH * HW)

    maxv, feat12 = pl.pallas_call(
        _heat_kernel,
        grid=(N // BS1,),
        in_specs=[
            pl.BlockSpec((BS1, NCH * HW), lambda i: (i, 0)),
            pl.BlockSpec((BS1, 4), lambda i: (i, 0)),
        ],
        out_specs=[
            pl.BlockSpec((BS1, 4), lambda i: (i, 0)),
            pl.BlockSpec((BS1, 12), lambda i: (i, 0)),
        ],
        out_shape=[
            jax.ShapeDtypeStruct((N, 4), jnp.float32),
            jax.ShapeDtypeStruct((N, 12), jnp.float32),
        ],
    )(pred2, det_bboxes)

    heat = jnp.mean(jax.nn.sigmoid(maxv), axis=1)
    scores = 0.5 * cls_scores + 0.5 * heat
    feat14 = jnp.concatenate(
        [feat12, scores[:, None], labels.astype(jnp.float32)[:, None]],
        axis=1)
    featT = feat14.T

    rank3 = pl.pallas_call(
        _rank_kernel,
        grid=(N // BS2,),
        in_specs=[
            pl.BlockSpec((BS2, 14), lambda i: (i, 0)),
            pl.BlockSpec((14, N), lambda i: (0, 0)),
        ],
        out_specs=pl.BlockSpec((1, 1, N), lambda i: (0, 0, 0)),
        out_shape=jax.ShapeDtypeStruct((1, 1, N), jnp.float32),
    )(feat14, featT)

    sd = pl.pallas_call(
        _permute_kernel,
        grid=(N // BS2,),
        in_specs=[
            pl.BlockSpec((1, 1, N), lambda i: (0, 0, 0)),
            pl.BlockSpec((N, 14), lambda i: (0, 0)),
        ],
        out_specs=pl.BlockSpec((BS2, 14), lambda i: (i, 0)),
        out_shape=jax.ShapeDtypeStruct((N, 14), jnp.float32),
    )(rank3, feat14)
    sdt = sd.T

    mt = pl.pallas_call(
        _mt_kernel,
        grid=(N // BS2,),
        in_specs=[
            pl.BlockSpec((BS2, 14), lambda i: (i, 0)),
            pl.BlockSpec((14, N), lambda i: (0, 0)),
        ],
        out_specs=pl.BlockSpec((BS2, N), lambda i: (i, 0)),
        out_shape=jax.ShapeDtypeStruct((N, N), jnp.float32),
    )(sd, sdt)

    out = pl.pallas_call(
        _fix_kernel,
        grid=(1,),
        in_specs=[
            pl.BlockSpec((N, N), lambda i: (0, 0)),
            pl.BlockSpec((N, 14), lambda i: (0, 0)),
        ],
        out_specs=pl.BlockSpec((N, 9), lambda i: (0, 0)),
        out_shape=jax.ShapeDtypeStruct((N, 9), jnp.float32),
    )(mt, sd)
    return out


# speedup vs baseline: 52.8338x; 1.1411x over previous
import jax
import jax.numpy as jnp
from jax.experimental import pallas as pl

N = 2000
CH = 5
HMAP = 56
HW = HMAP * HMAP
NCH = 4
TH = 0.2
DET = 150

BS1 = 200
BS2 = 400


def _heat_kernel(pred_ref, det_ref, maxv_ref, feat_ref):
    det = det_ref[...]
    ms, idxs = [], []
    for c in range(NCH):
        xc = pred_ref[:, c * HW:(c + 1) * HW]
        mc = jnp.max(xc, axis=1, keepdims=True)
        pos = jax.lax.broadcasted_iota(jnp.int32, xc.shape, 1)
        idxc = jnp.min(jnp.where(xc == mc, pos, HW), axis=1, keepdims=True)
        ms.append(mc)
        idxs.append(idxc)
    m = jnp.concatenate(ms, axis=1)
    idx = jnp.concatenate(idxs, axis=1)
    xsf = (idx % HMAP).astype(jnp.float32)
    ysf = (idx // HMAP).astype(jnp.float32)
    widths = det[:, 2:3] - det[:, 0:1]
    heights = det[:, 3:4] - det[:, 1:2]
    x1 = det[:, 0:1] - widths / 2
    y1 = det[:, 1:2] - heights / 2
    abs_xs = (xsf + 0.5) / HMAP * widths * 2 + x1
    abs_ys = (ysf + 0.5) / HMAP * heights * 2 + y1
    bx1 = jnp.min(abs_xs, axis=1, keepdims=True)
    bx2 = jnp.max(abs_xs, axis=1, keepdims=True)
    by1 = jnp.min(abs_ys, axis=1, keepdims=True)
    by2 = jnp.max(abs_ys, axis=1, keepdims=True)
    maxv_ref[...] = m
    feat_ref[...] = jnp.concatenate(
        [abs_xs, abs_ys, bx1, bx2, by1, by2], axis=1)


def _nms_kernel(feat_ref, featT_ref, out_ref):
    feat = feat_ref[...]
    s_col = feat[:, 12:13]
    s_row = featT_ref[12:13, :]
    jr = jax.lax.broadcasted_iota(jnp.int32, (1, N), 1)

    rank_blocks = []
    for b in range(N // BS2):
        s_i = s_col[b * BS2:(b + 1) * BS2, :]
        ic = (jax.lax.broadcasted_iota(jnp.int32, (BS2, 1), 0) + b * BS2)
        beats = (s_row > s_i) | ((s_row == s_i) & (jr < ic))
        rank_blocks.append(
            jnp.sum(jnp.where(beats, 1.0, 0.0), axis=1, keepdims=True))
    rank_col = jnp.concatenate(rank_blocks, axis=0)
    rank_row = jnp.transpose(rank_col)

    sd_blocks = []
    for b in range(N // BS2):
        p = (jax.lax.broadcasted_iota(jnp.int32, (BS2, 1), 0)
             + b * BS2).astype(jnp.float32)
        onehot = jnp.where(rank_row == p, 1.0, 0.0)
        sd_blocks.append(
            jnp.dot(onehot, feat, preferred_element_type=jnp.float32))
    sd = jnp.concatenate(sd_blocks, axis=0)
    sdt = jnp.transpose(sd)

    bx1j = sdt[8:9, :]
    bx2j = sdt[9:10, :]
    by1j = sdt[10:11, :]
    by2j = sdt[11:12, :]
    labj = sdt[13:14, :]
    aj = jnp.maximum(bx2j - bx1j, 0.0) * jnp.maximum(by2j - by1j, 0.0)

    mt_blocks = []
    for b in range(N // BS2):
        sdb = sd[b * BS2:(b + 1) * BS2, :]
        bx1i, bx2i = sdb[:, 8:9], sdb[:, 9:10]
        by1i, by2i = sdb[:, 10:11], sdb[:, 11:12]
        labi = sdb[:, 13:14]
        ix1 = jnp.maximum(bx1i, bx1j)
        iy1 = jnp.maximum(by1i, by1j)
        ix2 = jnp.minimum(bx2i, bx2j)
        iy2 = jnp.minimum(by2i, by2j)
        inter = jnp.maximum(ix2 - ix1, 0.0) * jnp.maximum(iy2 - iy1, 0.0)
        ai = jnp.maximum(bx2i - bx1i, 0.0) * jnp.maximum(by2i - by1i, 0.0)
        union = ai + aj - inter
        iou = inter / jnp.maximum(union, 1e-9)
        ic = (jax.lax.broadcasted_iota(jnp.int32, (BS2, 1), 0) + b * BS2)
        sup = (iou > TH) & (labi == labj) & (jr < ic)
        mt_blocks.append(jnp.where(sup, 1.0, 0.0))
    mt = jnp.concatenate(mt_blocks, axis=0)

    s = sd[:, 12:13]
    lab = sd[:, 13:14]
    k0 = jnp.where(lab > 0.0, 1.0, 0.0)

    def step(k):
        amt = jnp.dot(mt, k, preferred_element_type=jnp.float32)
        return k0 * jnp.where(amt == 0.0, 1.0, 0.0)

    k1 = step(k0)

    def cond(c):
        return c[1]

    def body(c):
        k, _ = c
        k2 = step(k)
        return k2, jnp.any(k2 != k)

    k, _ = jax.lax.while_loop(cond, body, (k1, jnp.any(k1 != k0)))

    cnt_blocks = []
    for b in range(N // BS2):
        ic = (jax.lax.broadcasted_iota(jnp.int32, (BS2, 1), 0) + b * BS2)
        lle = jnp.where(jr <= ic, 1.0, 0.0)
        cnt_blocks.append(jnp.dot(lle, k, preferred_element_type=jnp.float32))
    cnt = jnp.concatenate(cnt_blocks, axis=0)
    sel = (k > 0.0) & (cnt == jnp.float32(DET))
    th = jnp.max(jnp.where(sel, s, -jnp.inf))
    kf = (k > 0.0) & (s >= th)
    fs = jnp.where(kf, s, 0.0)
    out_ref[...] = jnp.concatenate(
        [sd[:, 0:1], sd[:, 4:5], sd[:, 1:2], sd[:, 5:6],
         sd[:, 2:3], sd[:, 6:7], sd[:, 3:4], sd[:, 7:8], fs], axis=1)


def kernel(pred, det_bboxes, cls_scores, labels):
    pred2 = pred.reshape(N, CH * HW)

    maxv, feat12 = pl.pallas_call(
        _heat_kernel,
        grid=(N // BS1,),
        in_specs=[
            pl.BlockSpec((BS1, NCH * HW), lambda i: (i, 0)),
            pl.BlockSpec((BS1, 4), lambda i: (i, 0)),
        ],
        out_specs=[
            pl.BlockSpec((BS1, 4), lambda i: (i, 0)),
            pl.BlockSpec((BS1, 12), lambda i: (i, 0)),
        ],
        out_shape=[
            jax.ShapeDtypeStruct((N, 4), jnp.float32),
            jax.ShapeDtypeStruct((N, 12), jnp.float32),
        ],
    )(pred2, det_bboxes)

    heat = jnp.mean(jax.nn.sigmoid(maxv), axis=1)
    scores = 0.5 * cls_scores + 0.5 * heat
    feat14 = jnp.concatenate(
        [feat12, scores[:, None], labels.astype(jnp.float32)[:, None]],
        axis=1)
    featT = feat14.T

    out = pl.pallas_call(
        _nms_kernel,
        grid=(1,),
        in_specs=[
            pl.BlockSpec((N, 14), lambda i: (0, 0)),
            pl.BlockSpec((14, N), lambda i: (0, 0)),
        ],
        out_specs=pl.BlockSpec((N, 9), lambda i: (0, 0)),
        out_shape=jax.ShapeDtypeStruct((N, 9), jnp.float32),
    )(feat14, featT)
    return out
